# baseline (speedup 1.0000x reference)
import functools

import jax
import jax.numpy as jnp
from jax import lax
from jax.experimental import pallas as pl
from jax.experimental.pallas import tpu as pltpu
from jax.experimental.pallas import tpu_sc as plsc

N = 10000
E = 320000
D = 128

NP = 10240
NW = 32
EPW = E // NW
CW = 128
NF = EPW // CW
TAIL = EPW - NF * CW
HCF = NF // 2
RPT = NP // 16
BLK = 1024
GRID = NP // BLK

_mesh = plsc.VectorSubcoreMesh(core_axis_name="c", subcore_axis_name="s")


@functools.partial(
    pl.kernel,
    mesh=_mesh,
    out_type=jax.ShapeDtypeStruct((2 * NP,), jnp.float32),
    scratch_types=[
        pltpu.VMEM((EPW,), jnp.int32),
        pltpu.VMEM((NF + 1, CW), jnp.int32),
        pltpu.VMEM((CW,), jnp.float32),
        pltpu.VMEM((RPT,), jnp.float32),
        pltpu.VMEM_SHARED((NP,), jnp.float32),
        pltpu.SemaphoreType.DMA,
    ],
)
def _deg_kernel(edges_hbm, ones_hbm, zeros_hbm, cnt_hbm, idx1_v, idx_v,
                ones_v, buf_v, acc_sh, sem):
    cid = lax.axis_index("c")
    sid = lax.axis_index("s")
    wid = cid * 16 + sid
    row0 = sid * RPT

    pltpu.sync_copy(edges_hbm.at[pl.ds(E + wid * EPW, EPW)], idx1_v)
    pltpu.sync_copy(ones_hbm, ones_v)
    pltpu.sync_copy(zeros_hbm, buf_v)
    pltpu.sync_copy(buf_v, acc_sh.at[pl.ds(row0, RPT)])

    @pl.loop(0, NF)
    def _(j):
        @pl.loop(0, CW // 16)
        def _(k):
            idx_v[j, pl.ds(k * 16, 16)] = idx1_v[pl.ds(j * CW + k * 16, 16)]

    idx_v[NF, pl.ds(0, 16)] = idx1_v[pl.ds(NF * CW, 16)]

    @pl.loop(1, CW // 16)
    def _(k):
        idx_v[NF, pl.ds(k * 16, 16)] = jnp.full((16,), N + wid, jnp.int32)

    plsc.subcore_barrier()

    @pl.loop(0, NF + 1)
    def _(j):
        pltpu.async_copy(ones_v, acc_sh.at[idx_v.at[j]], sem, add=True)

    @pl.loop(0, NF + 1)
    def _(j):
        pltpu.make_async_copy(ones_v, acc_sh.at[idx_v.at[0]], sem).wait()

    plsc.subcore_barrier()
    pltpu.sync_copy(acc_sh.at[pl.ds(row0, RPT)], buf_v)
    pltpu.sync_copy(buf_v, cnt_hbm.at[pl.ds(cid * NP + row0, RPT)])


def _k2_body(c0_ref, c1_ref, x_ref, wct_ref, hn_ref):
    deg = 1.0 + c0_ref[...] + c1_ref[...]
    dis = lax.rsqrt(deg).reshape(BLK, 1)
    h = jnp.dot(x_ref[...], wct_ref[...], preferred_element_type=jnp.float32)
    hn_ref[...] = h * dis


_k2 = pl.pallas_call(
    _k2_body,
    grid=(GRID,),
    in_specs=[
        pl.BlockSpec((BLK,), lambda i: (i,)),
        pl.BlockSpec((BLK,), lambda i: (i,)),
        pl.BlockSpec((BLK, D), lambda i: (i, 0)),
        pl.BlockSpec((D, D), lambda i: (0, 0)),
    ],
    out_specs=pl.BlockSpec((BLK, D), lambda i: (i, 0)),
    out_shape=jax.ShapeDtypeStruct((NP, D), jnp.float32),
)


@functools.partial(
    pl.kernel,
    mesh=_mesh,
    out_type=[
        jax.ShapeDtypeStruct((NP, D), jnp.float32),
        jax.ShapeDtypeStruct((NP, D), jnp.float32),
    ],
    scratch_types=[
        pltpu.VMEM((HCF * CW,), jnp.int32),
        pltpu.VMEM((HCF * CW,), jnp.int32),
        pltpu.VMEM((2, CW), jnp.int32),
        pltpu.VMEM((CW,), jnp.int32),
        pltpu.VMEM((CW, D), jnp.float32),
        pltpu.VMEM((CW, D), jnp.float32),
        pltpu.VMEM_SHARED((NP, D), jnp.float32),
        pltpu.SemaphoreType.DMA,
        pltpu.SemaphoreType.DMA,
    ],
)
def _scatter_kernel(edges_hbm, hn_hbm, z2_hbm, p0_hbm, p1_hbm, sidx, didx,
                    dstage, gidxt, rows0, rows1, acc_sh, sem0, sem1):
    cid = lax.axis_index("c")
    sid = lax.axis_index("s")
    wid = cid * 16 + sid
    row0 = sid * RPT
    base = wid * EPW

    pltpu.sync_copy(z2_hbm, rows0)

    @pl.loop(0, RPT // CW)
    def _(k):
        pltpu.sync_copy(rows0, acc_sh.at[pl.ds(row0 + k * CW, CW)])

    plsc.subcore_barrier()

    def _stage(j, slot):
        @pl.loop(0, CW // 16)
        def _(k):
            dstage[slot, pl.ds(k * 16, 16)] = didx[pl.ds(j * CW + k * 16, 16)]

    def _gidx(j):
        return sidx.at[pl.ds(j * CW, CW)]

    @pl.loop(0, 2)
    def _(h):
        off = base + h * (HCF * CW)
        pltpu.sync_copy(edges_hbm.at[pl.ds(off, HCF * CW)], sidx)
        pltpu.sync_copy(edges_hbm.at[pl.ds(E + off, HCF * CW)], didx)

        _stage(0, 0)
        pltpu.async_copy(hn_hbm.at[_gidx(0)], rows0, sem0).wait()

        @pl.loop(0, HCF - 2, step=2)
        def _(j):
            cp1 = pltpu.async_copy(hn_hbm.at[_gidx(j + 1)], rows1, sem1)
            _stage(j + 1, 1)
            pltpu.sync_copy(rows0, acc_sh.at[dstage.at[0]], add=True)
            cp1.wait()
            cp0 = pltpu.async_copy(hn_hbm.at[_gidx(j + 2)], rows0, sem0)
            _stage(j + 2, 0)
            pltpu.sync_copy(rows1, acc_sh.at[dstage.at[1]], add=True)
            cp0.wait()

        _stage(HCF - 1, 0)
        pltpu.sync_copy(rows0, acc_sh.at[dstage.at[0]], add=True)

    toff = base + NF * CW
    pltpu.sync_copy(edges_hbm.at[pl.ds(toff, TAIL)], gidxt.at[pl.ds(0, TAIL)])
    pltpu.sync_copy(edges_hbm.at[pl.ds(E + toff, TAIL)],
                    didx.at[pl.ds(0, TAIL)])
    dstage[0, pl.ds(0, 16)] = didx[pl.ds(0, 16)]

    @pl.loop(1, CW // 16)
    def _(k):
        gidxt[pl.ds(k * 16, 16)] = jnp.full((16,), wid * 312, jnp.int32)
        dstage[0, pl.ds(k * 16, 16)] = jnp.full((16,), N + wid, jnp.int32)

    pltpu.sync_copy(hn_hbm.at[gidxt], rows0)
    pltpu.sync_copy(rows0, acc_sh.at[dstage.at[0]], add=True)

    plsc.subcore_barrier()

    @pl.loop(0, RPT // CW)
    def _(k):
        sl = pl.ds(row0 + k * CW, CW)
        pltpu.sync_copy(acc_sh.at[sl], rows0)

        @pl.when(cid == 0)
        def _():
            pltpu.sync_copy(rows0, p0_hbm.at[sl])

        @pl.when(cid == 1)
        def _():
            pltpu.sync_copy(rows0, p1_hbm.at[sl])


def _k4_body(p0_ref, p1_ref, hn_ref, c0_ref, c1_ref, w1t_ref, b1_ref,
             w2t_ref, b2_ref, y_ref, g_ref):
    s = p0_ref[...] + p1_ref[...] + hn_ref[...]
    deg = 1.0 + c0_ref[...] + c1_ref[...]
    dis = lax.rsqrt(deg).reshape(BLK, 1)
    g = jnp.maximum(dis * s, 0.0)
    h1 = jnp.maximum(
        jnp.dot(g, w1t_ref[...], preferred_element_type=jnp.float32)
        + b1_ref[...], 0.0)
    y = (jnp.dot(h1, w2t_ref[...], preferred_element_type=jnp.float32)
         + b2_ref[...])
    y_ref[...] = y.reshape(BLK, 1, D)
    g_ref[...] = g


_k4 = pl.pallas_call(
    _k4_body,
    grid=(GRID,),
    in_specs=[
        pl.BlockSpec((BLK, D), lambda i: (i, 0)),
        pl.BlockSpec((BLK, D), lambda i: (i, 0)),
        pl.BlockSpec((BLK, D), lambda i: (i, 0)),
        pl.BlockSpec((BLK,), lambda i: (i,)),
        pl.BlockSpec((BLK,), lambda i: (i,)),
        pl.BlockSpec((D, D), lambda i: (0, 0)),
        pl.BlockSpec((1, D), lambda i: (0, 0)),
        pl.BlockSpec((D, D), lambda i: (0, 0)),
        pl.BlockSpec((1, D), lambda i: (0, 0)),
    ],
    out_specs=[
        pl.BlockSpec((BLK, 1, D), lambda i: (i, 0, 0)),
        pl.BlockSpec((BLK, D), lambda i: (i, 0)),
    ],
    out_shape=[
        jax.ShapeDtypeStruct((N, 1, D), jnp.float32),
        jax.ShapeDtypeStruct((N, D), jnp.float32),
    ],
)


def kernel(x, edge_index, Wc, W1, b1, W2, b2):
    edges = edge_index.reshape(2 * E)

    ones1 = jnp.ones((CW,), jnp.float32)
    zeros1 = jnp.zeros((RPT,), jnp.float32)
    zeros2 = jnp.zeros((CW, D), jnp.float32)

    counts = _deg_kernel(edges, ones1, zeros1)
    c0 = counts[:NP]
    c1 = counts[NP:]

    hn = _k2(c0, c1, x, Wc.T)

    p0, p1 = _scatter_kernel(edges, hn, zeros2)

    y, g = _k4(p0, p1, hn, c0, c1, W1.T, b1[None, :], W2.T, b2[None, :])
    return (y, g)


# speedup vs baseline: 1.0507x; 1.0507x over previous
import functools

import jax
import jax.numpy as jnp
from jax import lax
from jax.experimental import pallas as pl
from jax.experimental.pallas import tpu as pltpu
from jax.experimental.pallas import tpu_sc as plsc

N = 10000
E = 320000
D = 128

NP = 10240
NW = 32
EPW = E // NW
CW = 128
NF = EPW // CW
TAIL = EPW - NF * CW
HCF = NF // 2
RPT = NP // 16
BLK = 1024
GRID = NP // BLK

_mesh = plsc.VectorSubcoreMesh(core_axis_name="c", subcore_axis_name="s")


@functools.partial(
    pl.kernel,
    mesh=_mesh,
    out_type=jax.ShapeDtypeStruct((2 * NP,), jnp.float32),
    scratch_types=[
        pltpu.VMEM((EPW,), jnp.int32),
        pltpu.VMEM((NF + 1, CW), jnp.int32),
        pltpu.VMEM((CW,), jnp.float32),
        pltpu.VMEM((RPT,), jnp.float32),
        pltpu.VMEM_SHARED((NP,), jnp.float32),
        pltpu.SemaphoreType.DMA,
    ],
)
def _deg_kernel(edges_hbm, ones_hbm, zeros_hbm, cnt_hbm, idx1_v, idx_v,
                ones_v, buf_v, acc_sh, sem):
    cid = lax.axis_index("c")
    sid = lax.axis_index("s")
    wid = cid * 16 + sid
    row0 = sid * RPT

    pltpu.sync_copy(edges_hbm.at[pl.ds(E + wid * EPW, EPW)], idx1_v)
    pltpu.sync_copy(ones_hbm, ones_v)
    pltpu.sync_copy(zeros_hbm, buf_v)
    pltpu.sync_copy(buf_v, acc_sh.at[pl.ds(row0, RPT)])

    @pl.loop(0, NF)
    def _(j):
        @pl.loop(0, CW // 16)
        def _(k):
            idx_v[j, pl.ds(k * 16, 16)] = idx1_v[pl.ds(j * CW + k * 16, 16)]

    idx_v[NF, pl.ds(0, 16)] = idx1_v[pl.ds(NF * CW, 16)]

    @pl.loop(1, CW // 16)
    def _(k):
        idx_v[NF, pl.ds(k * 16, 16)] = N + k * 16 + lax.iota(jnp.int32, 16)

    plsc.subcore_barrier()

    @pl.loop(0, NF + 1)
    def _(j):
        pltpu.async_copy(ones_v, acc_sh.at[idx_v.at[j]], sem, add=True)

    @pl.loop(0, NF + 1)
    def _(j):
        pltpu.make_async_copy(ones_v, acc_sh.at[idx_v.at[0]], sem).wait()

    plsc.subcore_barrier()
    pltpu.sync_copy(acc_sh.at[pl.ds(row0, RPT)], buf_v)
    pltpu.sync_copy(buf_v, cnt_hbm.at[pl.ds(cid * NP + row0, RPT)])


def _k2_body(c0_ref, c1_ref, x_ref, wct_ref, hn_ref):
    deg = 1.0 + c0_ref[...] + c1_ref[...]
    dis = lax.rsqrt(deg).reshape(BLK, 1)
    h = jnp.dot(x_ref[...], wct_ref[...], preferred_element_type=jnp.float32)
    hn_ref[...] = h * dis


_k2 = pl.pallas_call(
    _k2_body,
    grid=(GRID,),
    in_specs=[
        pl.BlockSpec((BLK,), lambda i: (i,)),
        pl.BlockSpec((BLK,), lambda i: (i,)),
        pl.BlockSpec((BLK, D), lambda i: (i, 0)),
        pl.BlockSpec((D, D), lambda i: (0, 0)),
    ],
    out_specs=pl.BlockSpec((BLK, D), lambda i: (i, 0)),
    out_shape=jax.ShapeDtypeStruct((NP, D), jnp.float32),
)


@functools.partial(
    pl.kernel,
    mesh=_mesh,
    out_type=[
        jax.ShapeDtypeStruct((NP, D), jnp.float32),
        jax.ShapeDtypeStruct((NP, D), jnp.float32),
    ],
    scratch_types=[
        pltpu.VMEM((HCF * CW,), jnp.int32),
        pltpu.VMEM((HCF * CW,), jnp.int32),
        pltpu.VMEM((2, CW), jnp.int32),
        pltpu.VMEM((CW,), jnp.int32),
        pltpu.VMEM((CW, D), jnp.float32),
        pltpu.VMEM((CW, D), jnp.float32),
        pltpu.VMEM_SHARED((NP, D), jnp.float32),
        pltpu.SemaphoreType.DMA,
        pltpu.SemaphoreType.DMA,
    ],
)
def _scatter_kernel(edges_hbm, hn_hbm, z2_hbm, p0_hbm, p1_hbm, sidx, didx,
                    dstage, gidxt, rows0, rows1, acc_sh, sem0, sem1):
    cid = lax.axis_index("c")
    sid = lax.axis_index("s")
    wid = cid * 16 + sid
    row0 = sid * RPT
    base = wid * EPW

    pltpu.sync_copy(z2_hbm, rows0)

    @pl.loop(0, RPT // CW)
    def _(k):
        pltpu.sync_copy(rows0, acc_sh.at[pl.ds(row0 + k * CW, CW)])

    plsc.subcore_barrier()

    def _stage(j, slot):
        @pl.loop(0, CW // 16)
        def _(k):
            dstage[slot, pl.ds(k * 16, 16)] = didx[pl.ds(j * CW + k * 16, 16)]

    def _gidx(j):
        return sidx.at[pl.ds(j * CW, CW)]

    @pl.loop(0, 2)
    def _(h):
        off = base + h * (HCF * CW)
        pltpu.sync_copy(edges_hbm.at[pl.ds(off, HCF * CW)], sidx)
        pltpu.sync_copy(edges_hbm.at[pl.ds(E + off, HCF * CW)], didx)

        _stage(0, 0)
        pltpu.async_copy(hn_hbm.at[_gidx(0)], rows0, sem0).wait()

        @pl.loop(0, HCF - 2, step=2)
        def _(j):
            cp1 = pltpu.async_copy(hn_hbm.at[_gidx(j + 1)], rows1, sem1)
            _stage(j + 1, 1)
            pltpu.sync_copy(rows0, acc_sh.at[dstage.at[0]], add=True)
            cp1.wait()
            cp0 = pltpu.async_copy(hn_hbm.at[_gidx(j + 2)], rows0, sem0)
            _stage(j + 2, 0)
            pltpu.sync_copy(rows1, acc_sh.at[dstage.at[1]], add=True)
            cp0.wait()

        _stage(HCF - 1, 0)
        pltpu.sync_copy(rows0, acc_sh.at[dstage.at[0]], add=True)

    toff = base + NF * CW
    pltpu.sync_copy(edges_hbm.at[pl.ds(toff, TAIL)], gidxt.at[pl.ds(0, TAIL)])
    pltpu.sync_copy(edges_hbm.at[pl.ds(E + toff, TAIL)],
                    didx.at[pl.ds(0, TAIL)])
    dstage[0, pl.ds(0, 16)] = didx[pl.ds(0, 16)]

    @pl.loop(1, CW // 16)
    def _(k):
        gidxt[pl.ds(k * 16, 16)] = sidx[pl.ds(k * 16, 16)]
        dstage[0, pl.ds(k * 16, 16)] = (
            N + k * 16 + lax.iota(jnp.int32, 16))

    pltpu.sync_copy(hn_hbm.at[gidxt], rows0)
    pltpu.sync_copy(rows0, acc_sh.at[dstage.at[0]], add=True)

    plsc.subcore_barrier()

    @pl.loop(0, RPT // CW)
    def _(k):
        sl = pl.ds(row0 + k * CW, CW)
        pltpu.sync_copy(acc_sh.at[sl], rows0)

        @pl.when(cid == 0)
        def _():
            pltpu.sync_copy(rows0, p0_hbm.at[sl])

        @pl.when(cid == 1)
        def _():
            pltpu.sync_copy(rows0, p1_hbm.at[sl])


def _k4_body(p0_ref, p1_ref, hn_ref, c0_ref, c1_ref, w1t_ref, b1_ref,
             w2t_ref, b2_ref, y_ref, g_ref):
    s = p0_ref[...] + p1_ref[...] + hn_ref[...]
    deg = 1.0 + c0_ref[...] + c1_ref[...]
    dis = lax.rsqrt(deg).reshape(BLK, 1)
    g = jnp.maximum(dis * s, 0.0)
    h1 = jnp.maximum(
        jnp.dot(g, w1t_ref[...], preferred_element_type=jnp.float32)
        + b1_ref[...], 0.0)
    y = (jnp.dot(h1, w2t_ref[...], preferred_element_type=jnp.float32)
         + b2_ref[...])
    y_ref[...] = y.reshape(BLK, 1, D)
    g_ref[...] = g


_k4 = pl.pallas_call(
    _k4_body,
    grid=(GRID,),
    in_specs=[
        pl.BlockSpec((BLK, D), lambda i: (i, 0)),
        pl.BlockSpec((BLK, D), lambda i: (i, 0)),
        pl.BlockSpec((BLK, D), lambda i: (i, 0)),
        pl.BlockSpec((BLK,), lambda i: (i,)),
        pl.BlockSpec((BLK,), lambda i: (i,)),
        pl.BlockSpec((D, D), lambda i: (0, 0)),
        pl.BlockSpec((1, D), lambda i: (0, 0)),
        pl.BlockSpec((D, D), lambda i: (0, 0)),
        pl.BlockSpec((1, D), lambda i: (0, 0)),
    ],
    out_specs=[
        pl.BlockSpec((BLK, 1, D), lambda i: (i, 0, 0)),
        pl.BlockSpec((BLK, D), lambda i: (i, 0)),
    ],
    out_shape=[
        jax.ShapeDtypeStruct((N, 1, D), jnp.float32),
        jax.ShapeDtypeStruct((N, D), jnp.float32),
    ],
)


def kernel(x, edge_index, Wc, W1, b1, W2, b2):
    edges = edge_index.reshape(2 * E)

    ones1 = jnp.ones((CW,), jnp.float32)
    zeros1 = jnp.zeros((RPT,), jnp.float32)
    zeros2 = jnp.zeros((CW, D), jnp.float32)

    counts = _deg_kernel(edges, ones1, zeros1)
    c0 = counts[:NP]
    c1 = counts[NP:]

    hn = _k2(c0, c1, x, Wc.T)

    p0, p1 = _scatter_kernel(edges, hn, zeros2)

    y, g = _k4(p0, p1, hn, c0, c1, W1.T, b1[None, :], W2.T, b2[None, :])
    return (y, g)


# speedup vs baseline: 1.0599x; 1.0088x over previous
import functools

import jax
import jax.numpy as jnp
from jax import lax
from jax.experimental import pallas as pl
from jax.experimental.pallas import tpu as pltpu
from jax.experimental.pallas import tpu_sc as plsc

N = 10000
E = 320000
D = 128

NP = 10240
NW = 32
EPW = E // NW
CW = 128
NF = EPW // CW
TAIL = EPW - NF * CW
HCF = NF // 2
RPT = NP // 16
BLK = 2048
GRID = NP // BLK

_mesh = plsc.VectorSubcoreMesh(core_axis_name="c", subcore_axis_name="s")


@functools.partial(
    pl.kernel,
    mesh=_mesh,
    out_type=jax.ShapeDtypeStruct((2 * NP,), jnp.float32),
    scratch_types=[
        pltpu.VMEM((EPW,), jnp.int32),
        pltpu.VMEM((NF + 1, CW), jnp.int32),
        pltpu.VMEM((CW,), jnp.float32),
        pltpu.VMEM((RPT,), jnp.float32),
        pltpu.VMEM_SHARED((NP,), jnp.float32),
        pltpu.SemaphoreType.DMA,
    ],
)
def _deg_kernel(edges_hbm, ones_hbm, zeros_hbm, cnt_hbm, idx1_v, idx_v,
                ones_v, buf_v, acc_sh, sem):
    cid = lax.axis_index("c")
    sid = lax.axis_index("s")
    wid = cid * 16 + sid
    row0 = sid * RPT

    pltpu.sync_copy(edges_hbm.at[pl.ds(E + wid * EPW, EPW)], idx1_v)
    pltpu.sync_copy(ones_hbm, ones_v)
    pltpu.sync_copy(zeros_hbm, buf_v)
    pltpu.sync_copy(buf_v, acc_sh.at[pl.ds(row0, RPT)])

    @pl.loop(0, NF)
    def _(j):
        @pl.loop(0, CW // 16)
        def _(k):
            idx_v[j, pl.ds(k * 16, 16)] = idx1_v[pl.ds(j * CW + k * 16, 16)]

    idx_v[NF, pl.ds(0, 16)] = idx1_v[pl.ds(NF * CW, 16)]

    @pl.loop(1, CW // 16)
    def _(k):
        idx_v[NF, pl.ds(k * 16, 16)] = N + k * 16 + lax.iota(jnp.int32, 16)

    plsc.subcore_barrier()

    @pl.loop(0, NF + 1)
    def _(j):
        pltpu.async_copy(ones_v, acc_sh.at[idx_v.at[j]], sem, add=True)

    @pl.loop(0, NF + 1)
    def _(j):
        pltpu.make_async_copy(ones_v, acc_sh.at[idx_v.at[0]], sem).wait()

    plsc.subcore_barrier()
    pltpu.sync_copy(acc_sh.at[pl.ds(row0, RPT)], buf_v)
    pltpu.sync_copy(buf_v, cnt_hbm.at[pl.ds(cid * NP + row0, RPT)])


def _k2a_body(x_ref, wct_ref, h_ref):
    h_ref[...] = jnp.dot(x_ref[...], wct_ref[...],
                         preferred_element_type=jnp.float32)


_k2a = pl.pallas_call(
    _k2a_body,
    grid=(GRID,),
    in_specs=[
        pl.BlockSpec((BLK, D), lambda i: (i, 0)),
        pl.BlockSpec((D, D), lambda i: (0, 0)),
    ],
    out_specs=pl.BlockSpec((BLK, D), lambda i: (i, 0)),
    out_shape=jax.ShapeDtypeStruct((NP, D), jnp.float32),
)


def _k2b_body(h_ref, c0_ref, c1_ref, hn_ref):
    deg = 1.0 + c0_ref[...] + c1_ref[...]
    dis = lax.rsqrt(deg).reshape(BLK, 1)
    hn_ref[...] = h_ref[...] * dis


_k2b = pl.pallas_call(
    _k2b_body,
    grid=(GRID,),
    in_specs=[
        pl.BlockSpec((BLK, D), lambda i: (i, 0)),
        pl.BlockSpec((BLK,), lambda i: (i,)),
        pl.BlockSpec((BLK,), lambda i: (i,)),
    ],
    out_specs=pl.BlockSpec((BLK, D), lambda i: (i, 0)),
    out_shape=jax.ShapeDtypeStruct((NP, D), jnp.float32),
)


@functools.partial(
    pl.kernel,
    mesh=_mesh,
    out_type=[
        jax.ShapeDtypeStruct((NP, D), jnp.float32),
        jax.ShapeDtypeStruct((NP, D), jnp.float32),
    ],
    scratch_types=[
        pltpu.VMEM((HCF * CW,), jnp.int32),
        pltpu.VMEM((HCF * CW,), jnp.int32),
        pltpu.VMEM((2, CW), jnp.int32),
        pltpu.VMEM((CW,), jnp.int32),
        pltpu.VMEM((CW, D), jnp.float32),
        pltpu.VMEM((CW, D), jnp.float32),
        pltpu.VMEM_SHARED((NP, D), jnp.float32),
        pltpu.SemaphoreType.DMA,
        pltpu.SemaphoreType.DMA,
    ],
)
def _scatter_kernel(edges_hbm, hn_hbm, z2_hbm, p0_hbm, p1_hbm, sidx, didx,
                    dstage, gidxt, rows0, rows1, acc_sh, sem0, sem1):
    cid = lax.axis_index("c")
    sid = lax.axis_index("s")
    wid = cid * 16 + sid
    row0 = sid * RPT
    base = wid * EPW

    pltpu.sync_copy(z2_hbm, rows0)

    @pl.loop(0, RPT // CW)
    def _(k):
        pltpu.sync_copy(rows0, acc_sh.at[pl.ds(row0 + k * CW, CW)])

    plsc.subcore_barrier()

    def _stage(j, slot):
        @pl.loop(0, CW // 16)
        def _(k):
            dstage[slot, pl.ds(k * 16, 16)] = didx[pl.ds(j * CW + k * 16, 16)]

    def _gidx(j):
        return sidx.at[pl.ds(j * CW, CW)]

    @pl.loop(0, 2)
    def _(h):
        off = base + h * (HCF * CW)
        pltpu.sync_copy(edges_hbm.at[pl.ds(off, HCF * CW)], sidx)
        pltpu.sync_copy(edges_hbm.at[pl.ds(E + off, HCF * CW)], didx)

        _stage(0, 0)
        pltpu.async_copy(hn_hbm.at[_gidx(0)], rows0, sem0).wait()

        @pl.loop(0, HCF - 2, step=2)
        def _(j):
            cp1 = pltpu.async_copy(hn_hbm.at[_gidx(j + 1)], rows1, sem1)
            _stage(j + 1, 1)
            pltpu.sync_copy(rows0, acc_sh.at[dstage.at[0]], add=True)
            cp1.wait()
            cp0 = pltpu.async_copy(hn_hbm.at[_gidx(j + 2)], rows0, sem0)
            _stage(j + 2, 0)
            pltpu.sync_copy(rows1, acc_sh.at[dstage.at[1]], add=True)
            cp0.wait()

        _stage(HCF - 1, 0)
        pltpu.sync_copy(rows0, acc_sh.at[dstage.at[0]], add=True)

    toff = base + NF * CW
    pltpu.sync_copy(edges_hbm.at[pl.ds(toff, TAIL)], gidxt.at[pl.ds(0, TAIL)])
    pltpu.sync_copy(edges_hbm.at[pl.ds(E + toff, TAIL)],
                    didx.at[pl.ds(0, TAIL)])
    dstage[0, pl.ds(0, 16)] = didx[pl.ds(0, 16)]

    @pl.loop(1, CW // 16)
    def _(k):
        gidxt[pl.ds(k * 16, 16)] = sidx[pl.ds(k * 16, 16)]
        dstage[0, pl.ds(k * 16, 16)] = (
            N + k * 16 + lax.iota(jnp.int32, 16))

    pltpu.sync_copy(hn_hbm.at[gidxt], rows0)
    pltpu.sync_copy(rows0, acc_sh.at[dstage.at[0]], add=True)

    plsc.subcore_barrier()

    @pl.loop(0, RPT // CW)
    def _(k):
        sl = pl.ds(row0 + k * CW, CW)
        pltpu.sync_copy(acc_sh.at[sl], rows0)

        @pl.when(cid == 0)
        def _():
            pltpu.sync_copy(rows0, p0_hbm.at[sl])

        @pl.when(cid == 1)
        def _():
            pltpu.sync_copy(rows0, p1_hbm.at[sl])


def _k4_body(p0_ref, p1_ref, hn_ref, c0_ref, c1_ref, w1t_ref, b1_ref,
             w2t_ref, b2_ref, y_ref, g_ref):
    s = p0_ref[...] + p1_ref[...] + hn_ref[...]
    deg = 1.0 + c0_ref[...] + c1_ref[...]
    dis = lax.rsqrt(deg).reshape(BLK, 1)
    g = jnp.maximum(dis * s, 0.0)
    h1 = jnp.maximum(
        jnp.dot(g, w1t_ref[...], preferred_element_type=jnp.float32)
        + b1_ref[...], 0.0)
    y = (jnp.dot(h1, w2t_ref[...], preferred_element_type=jnp.float32)
         + b2_ref[...])
    y_ref[...] = y.reshape(BLK, 1, D)
    g_ref[...] = g


_k4 = pl.pallas_call(
    _k4_body,
    grid=(GRID,),
    in_specs=[
        pl.BlockSpec((BLK, D), lambda i: (i, 0)),
        pl.BlockSpec((BLK, D), lambda i: (i, 0)),
        pl.BlockSpec((BLK, D), lambda i: (i, 0)),
        pl.BlockSpec((BLK,), lambda i: (i,)),
        pl.BlockSpec((BLK,), lambda i: (i,)),
        pl.BlockSpec((D, D), lambda i: (0, 0)),
        pl.BlockSpec((1, D), lambda i: (0, 0)),
        pl.BlockSpec((D, D), lambda i: (0, 0)),
        pl.BlockSpec((1, D), lambda i: (0, 0)),
    ],
    out_specs=[
        pl.BlockSpec((BLK, 1, D), lambda i: (i, 0, 0)),
        pl.BlockSpec((BLK, D), lambda i: (i, 0)),
    ],
    out_shape=[
        jax.ShapeDtypeStruct((N, 1, D), jnp.float32),
        jax.ShapeDtypeStruct((N, D), jnp.float32),
    ],
)


def kernel(x, edge_index, Wc, W1, b1, W2, b2):
    edges = edge_index.reshape(2 * E)

    ones1 = jnp.ones((CW,), jnp.float32)
    zeros1 = jnp.zeros((RPT,), jnp.float32)
    zeros2 = jnp.zeros((CW, D), jnp.float32)

    counts = _deg_kernel(edges, ones1, zeros1)
    c0 = counts[:NP]
    c1 = counts[NP:]

    h = _k2a(x, Wc.T)
    hn = _k2b(h, c0, c1)

    p0, p1 = _scatter_kernel(edges, hn, zeros2)

    y, g = _k4(p0, p1, hn, c0, c1, W1.T, b1[None, :], W2.T, b2[None, :])
    return (y, g)
